# baseline (speedup 1.0000x reference)
import jax
import jax.numpy as jnp
from jax.experimental import pallas as pl
from jax.experimental.pallas import tpu as pltpu

_DEPTH = 4
_T = 128


def _rq_kernel(x_ref, emb_ref, esq_ref, zsq0_ref,
               quants_ref, codes_ref, loss_ref):
    x = x_ref[...]
    emb = emb_ref[...]
    esq = esq_ref[...]
    K = emb.shape[0]

    lane = jax.lax.broadcasted_iota(jnp.int32, (1, K), 1).astype(jnp.float32)

    r = x
    agg = jnp.zeros_like(x)
    codes = []
    quants = []
    loss_vec = jnp.zeros((1, 128), dtype=jnp.float32)
    lane128 = jax.lax.broadcasted_iota(jnp.int32, (1, 128), 1)

    for d in range(_DEPTH):
        if d == 0:
            zsq = zsq0_ref[...]
        else:
            zsq = jnp.sum(r * r, axis=1, keepdims=True)
        dot = jax.lax.dot_general(
            r, emb, (((1,), (1,)), ((), ())),
            preferred_element_type=jnp.float32)
        s = (zsq + esq) - 2.0 * dot
        m = jnp.min(s, axis=1, keepdims=True)
        idxf = jnp.min(jnp.where(s == m, lane, float(K)),
                       axis=1, keepdims=True)
        onehot = (lane == idxf).astype(jnp.float32)
        q = jax.lax.dot_general(
            onehot, emb, (((1,), (0,)), ((), ())),
            preferred_element_type=jnp.float32,
            precision=jax.lax.Precision.HIGHEST)
        r = r - q
        agg = agg + q
        quants.append(agg)
        codes.append(idxf.astype(jnp.int32))
        part = jnp.sum((x - agg) * (x - agg))
        loss_vec = jnp.where(lane128 == d, part, loss_vec)

    quants_ref[...] = jnp.stack(quants, axis=0)
    codes_ref[...] = jnp.concatenate(codes, axis=1)
    loss_ref[...] = loss_vec.reshape(1, 1, 128)


def kernel(x, codebook):
    B, H, W, D = x.shape
    N = B * H * W
    flat = x.reshape(N, D)
    emb = codebook[:-1]
    K = emb.shape[0]
    esq = (emb * emb).sum(axis=-1)[None, :]
    zsq0 = (flat * flat).sum(axis=-1)[:, None]

    nt = N // _T
    quants, codes, loss_parts = pl.pallas_call(
        _rq_kernel,
        grid=(nt,),
        in_specs=[
            pl.BlockSpec((_T, D), lambda i: (i, 0)),
            pl.BlockSpec((K, D), lambda i: (0, 0)),
            pl.BlockSpec((1, K), lambda i: (0, 0)),
            pl.BlockSpec((_T, 1), lambda i: (i, 0)),
        ],
        out_specs=[
            pl.BlockSpec((_DEPTH, _T, D), lambda i: (0, i, 0)),
            pl.BlockSpec((_T, _DEPTH), lambda i: (i, 0)),
            pl.BlockSpec((1, 1, 128), lambda i: (i, 0, 0)),
        ],
        out_shape=[
            jax.ShapeDtypeStruct((_DEPTH, N, D), jnp.float32),
            jax.ShapeDtypeStruct((N, _DEPTH), jnp.int32),
            jax.ShapeDtypeStruct((nt, 1, 128), jnp.float32),
        ],
    )(flat, emb, esq, zsq0)

    n_el = float(N * D)
    sums = jnp.sum(loss_parts[:, 0, :_DEPTH], axis=0)
    commitment_loss = jnp.mean(sums / n_el)
    q_last = quants[_DEPTH - 1].reshape(x.shape)
    quants_trunc = x + jax.lax.stop_gradient(q_last - x)
    codes_out = codes.reshape(B, H, W, _DEPTH)
    return quants_trunc, commitment_loss, codes_out


# speedup vs baseline: 1.0124x; 1.0124x over previous
import jax
import jax.numpy as jnp
from jax.experimental import pallas as pl
from jax.experimental.pallas import tpu as pltpu

_DEPTH = 4
_T = 128


def _rq_kernel(x_ref, emb_ref, esq_ref, zsq0_ref,
               quants_ref, codes_ref, loss_ref):
    x = x_ref[...]
    K = emb_ref.shape[0]
    C = 2048

    lane_c = jax.lax.broadcasted_iota(jnp.int32, (1, C), 1).astype(jnp.float32)

    r = x
    agg = jnp.zeros_like(x)
    codes = []
    quants = []
    loss_vec = jnp.zeros((1, 128), dtype=jnp.float32)
    lane128 = jax.lax.broadcasted_iota(jnp.int32, (1, 128), 1)

    for d in range(_DEPTH):
        if d == 0:
            zsq = zsq0_ref[...]
        else:
            zsq = jnp.sum(r * r, axis=1, keepdims=True)
        run_m = jnp.full((x.shape[0], 1), jnp.inf, jnp.float32)
        run_i = jnp.zeros((x.shape[0], 1), jnp.float32)
        for j in range(K // C):
            e_j = emb_ref[pl.ds(j * C, C), :]
            esq_j = esq_ref[:, pl.ds(j * C, C)]
            dot = jax.lax.dot_general(
                r, e_j, (((1,), (1,)), ((), ())),
                preferred_element_type=jnp.float32)
            s = (zsq + esq_j) - 2.0 * dot
            m_j = jnp.min(s, axis=1, keepdims=True)
            i_j = jnp.min(jnp.where(s == m_j, lane_c + float(j * C), float(K)),
                          axis=1, keepdims=True)
            upd = m_j < run_m
            run_i = jnp.where(upd, i_j, run_i)
            run_m = jnp.where(upd, m_j, run_m)
        q = jnp.zeros_like(x)
        for j in range(K // C):
            e_j = emb_ref[pl.ds(j * C, C), :]
            onehot = (lane_c + float(j * C) == run_i).astype(jnp.float32)
            q = q + jax.lax.dot_general(
                onehot, e_j, (((1,), (0,)), ((), ())),
                preferred_element_type=jnp.float32,
                precision=jax.lax.Precision.HIGHEST)
        r = r - q
        agg = agg + q
        quants.append(agg)
        codes.append(run_i.astype(jnp.int32))
        part = jnp.sum((x - agg) * (x - agg))
        loss_vec = jnp.where(lane128 == d, part, loss_vec)

    quants_ref[...] = jnp.stack(quants, axis=0)
    codes_ref[...] = jnp.concatenate(codes, axis=1)
    loss_ref[...] = loss_vec.reshape(1, 1, 128)


def kernel(x, codebook):
    B, H, W, D = x.shape
    N = B * H * W
    flat = x.reshape(N, D)
    emb = codebook[:-1]
    K = emb.shape[0]
    esq = (emb * emb).sum(axis=-1)[None, :]
    zsq0 = (flat * flat).sum(axis=-1)[:, None]

    nt = N // _T
    quants, codes, loss_parts = pl.pallas_call(
        _rq_kernel,
        grid=(nt,),
        in_specs=[
            pl.BlockSpec((_T, D), lambda i: (i, 0)),
            pl.BlockSpec((K, D), lambda i: (0, 0)),
            pl.BlockSpec((1, K), lambda i: (0, 0)),
            pl.BlockSpec((_T, 1), lambda i: (i, 0)),
        ],
        out_specs=[
            pl.BlockSpec((_DEPTH, _T, D), lambda i: (0, i, 0)),
            pl.BlockSpec((_T, _DEPTH), lambda i: (i, 0)),
            pl.BlockSpec((1, 1, 128), lambda i: (i, 0, 0)),
        ],
        out_shape=[
            jax.ShapeDtypeStruct((_DEPTH, N, D), jnp.float32),
            jax.ShapeDtypeStruct((N, _DEPTH), jnp.int32),
            jax.ShapeDtypeStruct((nt, 1, 128), jnp.float32),
        ],
    )(flat, emb, esq, zsq0)

    n_el = float(N * D)
    sums = jnp.sum(loss_parts[:, 0, :_DEPTH], axis=0)
    commitment_loss = jnp.mean(sums / n_el)
    q_last = quants[_DEPTH - 1].reshape(x.shape)
    quants_trunc = x + jax.lax.stop_gradient(q_last - x)
    codes_out = codes.reshape(B, H, W, _DEPTH)
    return quants_trunc, commitment_loss, codes_out
